# baseline (speedup 1.0000x reference)
import functools

import jax
import jax.numpy as jnp
from jax import lax
from jax.experimental import pallas as pl

N = 4096
THRESHOLD = 1.0
ALPHA_PLUS = 0.01
INV_2TAU2 = 1.0 / (2.0 * 20.0 * 20.0)
BLOCK_R = 256


def _gstdp_block(spikes_ref, row_spikes_ref, w_ref, spikes_out_ref, w_out_ref):
    r = pl.program_id(0)
    s = spikes_ref[...]
    mask = (s >= THRESHOLD).astype(jnp.float32)
    spikes_out_ref[...] = mask
    n_spikes = jnp.sum(mask)

    row_mask = (row_spikes_ref[...] >= THRESHOLD).astype(jnp.float32)
    w = w_ref[...]

    col = lax.broadcasted_iota(jnp.int32, (BLOCK_R, N), 1)
    row = lax.broadcasted_iota(jnp.int32, (BLOCK_R, N), 0) + r * BLOCK_R
    d = (col - row).astype(jnp.float32)
    ltp = ALPHA_PLUS * jnp.exp(-(d * d) * INV_2TAU2)
    term = ltp * row_mask[:, None] * mask[None, :]
    term = jnp.where(col > row, term, 0.0)
    updated = jnp.clip(w + term, 0.0, 1.0)
    w_out_ref[...] = jnp.where(n_spikes > 1.0, updated, w)


@jax.jit
def kernel(input_spikes, weights):
    grid = (N // BLOCK_R,)
    spikes, new_w = pl.pallas_call(
        _gstdp_block,
        grid=grid,
        in_specs=[
            pl.BlockSpec((N,), lambda i: (0,)),
            pl.BlockSpec((BLOCK_R,), lambda i: (i,)),
            pl.BlockSpec((BLOCK_R, N), lambda i: (i, 0)),
        ],
        out_specs=[
            pl.BlockSpec((N,), lambda i: (0,)),
            pl.BlockSpec((BLOCK_R, N), lambda i: (i, 0)),
        ],
        out_shape=[
            jax.ShapeDtypeStruct((N,), jnp.float32),
            jax.ShapeDtypeStruct((N, N), jnp.float32),
        ],
    )(input_spikes, input_spikes, weights)
    return spikes, new_w


# speedup vs baseline: 1.3883x; 1.3883x over previous
import jax
import jax.numpy as jnp
from jax import lax
from jax.experimental import pallas as pl

N = 4096
THRESHOLD = 1.0
ALPHA_PLUS = 0.01
INV_2TAU2 = 1.0 / (2.0 * 20.0 * 20.0)
BLOCK_R = 256
BAND_W = 512


def _gstdp_block(spikes_ref, row_spikes_ref, w_ref, spikes_out_ref, w_out_ref):
    r = pl.program_id(0)
    s = spikes_ref[...]
    mask = (s >= THRESHOLD).astype(jnp.float32)
    spikes_out_ref[...] = mask
    many = jnp.sum(mask) > 1.0

    w = w_ref[...]
    w_out_ref[...] = jnp.where(many, jnp.clip(w, 0.0, 1.0), w)

    start = jnp.minimum(r * BLOCK_R, N - BAND_W)
    row_mask = (row_spikes_ref[...] >= THRESHOLD).astype(jnp.float32)
    col_mask = (spikes_ref[pl.ds(start, BAND_W)] >= THRESHOLD).astype(jnp.float32)
    wb = w_ref[:, pl.ds(start, BAND_W)]
    col = lax.broadcasted_iota(jnp.int32, (BLOCK_R, BAND_W), 1) + start
    row = lax.broadcasted_iota(jnp.int32, (BLOCK_R, BAND_W), 0) + r * BLOCK_R
    d = (col - row).astype(jnp.float32)
    ltp = ALPHA_PLUS * jnp.exp(-(d * d) * INV_2TAU2)
    term = ltp * row_mask[:, None] * col_mask[None, :]
    term = jnp.where(col > row, term, 0.0)
    updated = jnp.clip(wb + term, 0.0, 1.0)
    w_out_ref[:, pl.ds(start, BAND_W)] = jnp.where(many, updated, wb)


@jax.jit
def kernel(input_spikes, weights):
    grid = (N // BLOCK_R,)
    spikes, new_w = pl.pallas_call(
        _gstdp_block,
        grid=grid,
        in_specs=[
            pl.BlockSpec((N,), lambda i: (0,)),
            pl.BlockSpec((BLOCK_R,), lambda i: (i,)),
            pl.BlockSpec((BLOCK_R, N), lambda i: (i, 0)),
        ],
        out_specs=[
            pl.BlockSpec((N,), lambda i: (0,)),
            pl.BlockSpec((BLOCK_R, N), lambda i: (i, 0)),
        ],
        out_shape=[
            jax.ShapeDtypeStruct((N,), jnp.float32),
            jax.ShapeDtypeStruct((N, N), jnp.float32),
        ],
    )(input_spikes, input_spikes, weights)
    return spikes, new_w


# speedup vs baseline: 1.4151x; 1.0193x over previous
import jax
import jax.numpy as jnp
from jax import lax
from jax.experimental import pallas as pl

N = 4096
THRESHOLD = 1.0
ALPHA_PLUS = 0.01
INV_2TAU2 = 1.0 / (2.0 * 20.0 * 20.0)
BLOCK_R = 512
BAND_W = BLOCK_R + 128


def _gstdp_block(spikes_ref, row_spikes_ref, w_ref, spikes_out_ref, w_out_ref):
    r = pl.program_id(0)
    s = spikes_ref[...]
    mask = (s >= THRESHOLD).astype(jnp.float32)
    spikes_out_ref[...] = mask
    many = jnp.sum(mask) > 1.0

    @pl.when(jnp.logical_not(many))
    def _passthrough():
        w_out_ref[...] = w_ref[...]

    @pl.when(many)
    def _update():
        w = w_ref[...]
        w_out_ref[...] = jnp.clip(w, 0.0, 1.0)

        start = jnp.minimum(r * BLOCK_R, N - BAND_W)
        row_mask = (row_spikes_ref[...] >= THRESHOLD).astype(jnp.float32)
        col_mask = (spikes_ref[pl.ds(start, BAND_W)] >= THRESHOLD).astype(
            jnp.float32
        )
        wb = w_ref[:, pl.ds(start, BAND_W)]
        col = lax.broadcasted_iota(jnp.int32, (BLOCK_R, BAND_W), 1) + start
        row = lax.broadcasted_iota(jnp.int32, (BLOCK_R, BAND_W), 0) + r * BLOCK_R
        d = (col - row).astype(jnp.float32)
        ltp = ALPHA_PLUS * jnp.exp(-(d * d) * INV_2TAU2)
        term = ltp * row_mask[:, None] * col_mask[None, :]
        term = jnp.where(col > row, term, 0.0)
        w_out_ref[:, pl.ds(start, BAND_W)] = jnp.clip(wb + term, 0.0, 1.0)


@jax.jit
def kernel(input_spikes, weights):
    grid = (N // BLOCK_R,)
    spikes, new_w = pl.pallas_call(
        _gstdp_block,
        grid=grid,
        in_specs=[
            pl.BlockSpec((N,), lambda i: (0,)),
            pl.BlockSpec((BLOCK_R,), lambda i: (i,)),
            pl.BlockSpec((BLOCK_R, N), lambda i: (i, 0)),
        ],
        out_specs=[
            pl.BlockSpec((N,), lambda i: (0,)),
            pl.BlockSpec((BLOCK_R, N), lambda i: (i, 0)),
        ],
        out_shape=[
            jax.ShapeDtypeStruct((N,), jnp.float32),
            jax.ShapeDtypeStruct((N, N), jnp.float32),
        ],
    )(input_spikes, input_spikes, weights)
    return spikes, new_w
